# Initial kernel scaffold; baseline (speedup 1.0000x reference)
import jax
import jax.numpy as jnp
from jax.experimental import pallas as pl


def kernel(movie_genre, edge_index, user_emb, W_mp, b_mp, Wself_rates, Wneigh_rates, b_rates, Wself_rev, Wneigh_rev, b_rev, gamma_u, beta_u, gamma_m, beta_m):
    raise NotImplementedError("write your pallas kernel here")


# speedup vs baseline: 3.2725x; 3.2725x over previous
import jax
import jax.numpy as jnp
from jax import lax
from jax.experimental import pallas as pl
from jax.experimental.pallas import tpu as pltpu
from jax.experimental.pallas import tpu_sc as plsc

NU = 10000
NM = 10000
E = 320000
H = 128
MOVIE_IN = 32
EPS = 1e-5
SLOPE = 0.1

_NS = 16
_CHUNK = 128
_EPAD = 321536
_PER_TILE = _EPAD // _NS
_STEPS = _PER_TILE // _CHUNK
_ACC_ROWS = 10240
_TRASH = 10000
_ZROWS = _ACC_ROWS // _NS
_SROWS = 64


def _zero_shared(stage, acc, sid, width):
    z = jnp.zeros((16,), jnp.float32)

    def zrow(r, c0):
        for c in range(width // 16):
            stage[r, pl.ds(c * 16, 16)] = z
        return c0

    lax.fori_loop(0, _SROWS, zrow, 0)

    def zcp(j, c0):
        pltpu.sync_copy(stage, acc.at[pl.ds(sid * _ZROWS + j * _SROWS, _SROWS)])
        return c0

    lax.fori_loop(0, _ZROWS // _SROWS, zcp, 0)


def _copy_out(stage, acc, out, sid):
    def ocp(j, c0):
        off = sid * _ZROWS + j * _SROWS
        pltpu.sync_copy(acc.at[pl.ds(off, _SROWS)], stage)
        pltpu.sync_copy(stage, out.at[pl.ds(off, _SROWS)])
        return c0

    lax.fori_loop(0, _ZROWS // _SROWS, ocp, 0)


def _edge_body(hu, hm, src_g, dst_g, src_s, dst_s, sum_m, sum_u,
               idx_g, idx_s, rows, stage, acc, sem):
    cid = lax.axis_index("c")
    sid = lax.axis_index("s")
    _zero_shared(stage, acc, sid, H)
    plsc.subcore_barrier()

    def run(tbl, garr, sarr, out):
        base = sid * _PER_TILE

        def step(i, c0):
            off = base + i * _CHUNK
            pltpu.sync_copy(garr.at[pl.ds(off, _CHUNK)], idx_g)
            pltpu.sync_copy(sarr.at[pl.ds(off, _CHUNK)], idx_s)
            pltpu.async_copy(tbl.at[idx_g], rows, sem).wait()
            pltpu.sync_copy(rows, acc.at[idx_s], add=True)
            return c0

        lax.fori_loop(0, _STEPS, step, 0)
        plsc.subcore_barrier()
        _copy_out(stage, acc, out, sid)

    @pl.when(cid == 0)
    def _():
        run(hu, src_g, dst_s, sum_m)

    @pl.when(cid == 1)
    def _():
        run(hm, dst_g, src_s, sum_u)


import functools


@functools.cache
def _sc_calls():
    mesh = plsc.VectorSubcoreMesh(core_axis_name="c", subcore_axis_name="s",
                                  num_cores=2, num_subcores=_NS)
    edge_call = pl.kernel(
        _edge_body,
        out_type=(jax.ShapeDtypeStruct((_ACC_ROWS, H), jnp.float32),
                  jax.ShapeDtypeStruct((_ACC_ROWS, H), jnp.float32)),
        mesh=mesh,
        scratch_types=(
            pltpu.VMEM((_CHUNK,), jnp.int32),
            pltpu.VMEM((_CHUNK,), jnp.int32),
            pltpu.VMEM((_CHUNK, H), jnp.float32),
            pltpu.VMEM((_SROWS, H), jnp.float32),
            pltpu.VMEM_SHARED((_ACC_ROWS, H), jnp.float32),
            pltpu.SemaphoreType.DMA,
        ),
    )
    return edge_call


def _proj_body(mg, w, b, out):
    out[...] = (jnp.dot(mg[...], w[...], preferred_element_type=jnp.float32)
                + b[...][None, :])


_proj_call = pl.pallas_call(
    _proj_body,
    out_shape=jax.ShapeDtypeStruct((NM, H), jnp.float32),
)


def _bn_leaky(x, g, b):
    mu = jnp.mean(x, axis=0, keepdims=True)
    va = jnp.mean((x - mu) ** 2, axis=0, keepdims=True)
    y = (x - mu) * lax.rsqrt(va + EPS) * g[None, :] + b[None, :]
    return jnp.where(y > 0, y, SLOPE * y)


def _dense_body(hu, hm, summ, sumu, degm, degu,
                wsr, wnr, br, wsv, wnv, bv, gm, bm, gu, bu, huo, hmo):
    dm = jnp.maximum(degm[0:NM, 0:1], 1.0)
    du = jnp.maximum(degu[0:NU, 0:1], 1.0)
    neigh_m = summ[0:NM, :] / dm
    neigh_u = sumu[0:NU, :] / du
    new_m = (jnp.dot(hm[...], wsr[...], preferred_element_type=jnp.float32)
             + jnp.dot(neigh_m, wnr[...], preferred_element_type=jnp.float32)
             + br[...][None, :])
    new_u = (jnp.dot(hu[...], wsv[...], preferred_element_type=jnp.float32)
             + jnp.dot(neigh_u, wnv[...], preferred_element_type=jnp.float32)
             + bv[...][None, :])
    hmo[...] = hm[...] + _bn_leaky(new_m, gm[...], bm[...])
    huo[...] = hu[...] + _bn_leaky(new_u, gu[...], bu[...])


_dense_call = pl.pallas_call(
    _dense_body,
    out_shape=(jax.ShapeDtypeStruct((NU, H), jnp.float32),
               jax.ShapeDtypeStruct((NM, H), jnp.float32)),
)


def kernel(movie_genre, edge_index, user_emb, W_mp, b_mp,
           Wself_rates, Wneigh_rates, b_rates,
           Wself_rev, Wneigh_rev, b_rev,
           gamma_u, beta_u, gamma_m, beta_m):
    src = edge_index[0].astype(jnp.int32)
    dst = edge_index[1].astype(jnp.int32)
    pad = _EPAD - E
    zpad = jnp.zeros((pad,), jnp.int32)
    tpad = jnp.full((pad,), _TRASH, jnp.int32)
    src_g = jnp.concatenate([src, zpad])
    dst_g = jnp.concatenate([dst, zpad])
    src_s = jnp.concatenate([src, tpad])
    dst_s = jnp.concatenate([dst, tpad])

    _edge_call = _sc_calls()
    h_movie0 = _proj_call(movie_genre, W_mp, b_mp)
    ones_tab = jnp.ones((NU, H), jnp.float32)

    def _step(l, carry):
        h_user, h_movie, deg_m, deg_u = carry
        is_deg = l == 0
        tu = lax.cond(is_deg, lambda: ones_tab, lambda: h_user)
        tm = lax.cond(is_deg, lambda: ones_tab, lambda: h_movie)
        sum_m, sum_u = _edge_call(tu, tm, src_g, dst_g, src_s, dst_s)

        def deg_case():
            return h_user, h_movie, sum_m[:, 0:16], sum_u[:, 0:16]

        def layer_case():
            j = l - 1
            idx = lambda a: lax.dynamic_index_in_dim(a, j, 0, keepdims=False)
            hu2, hm2 = _dense_call(
                h_user, h_movie, sum_m, sum_u, deg_m, deg_u,
                idx(Wself_rates), idx(Wneigh_rates), idx(b_rates),
                idx(Wself_rev), idx(Wneigh_rev), idx(b_rev),
                idx(gamma_m), idx(beta_m), idx(gamma_u), idx(beta_u))
            return hu2, hm2, deg_m, deg_u

        return lax.cond(is_deg, deg_case, layer_case)

    zdeg = jnp.zeros((_ACC_ROWS, 16), jnp.float32)
    h_user, h_movie, _, _ = lax.fori_loop(
        0, 3, _step, (user_emb, h_movie0, zdeg, zdeg))
    return (h_user, h_movie)
